# baseline (speedup 1.0000x reference)
import functools

import jax
import jax.numpy as jnp
from jax import lax
from jax.experimental import pallas as pl
from jax.experimental.pallas import tpu as pltpu
from jax.experimental.pallas import tpu_sc as plsc

_EMBED = 1024
_E = 16
_B = 128

_NW = 32
_SC_E = 6
_TC_E = _E - _SC_E
_ROWS = _SC_E * _EMBED
_RPW = _ROWS // _NW
_CHUNK = 32
_RING = 3
_NCHUNK = _RPW // _CHUNK
_LANES = 16


def _rowsum_sc(W_flat):
    mesh = plsc.VectorSubcoreMesh(core_axis_name="c", subcore_axis_name="s")

    @functools.partial(
        pl.kernel,
        mesh=mesh,
        out_type=jax.ShapeDtypeStruct((_ROWS,), jnp.float32),
        scratch_types=[
            [pltpu.VMEM((_CHUNK, _EMBED), jnp.float32)] * _RING,
            pltpu.VMEM((_RPW,), jnp.float32),
            pltpu.VMEM((_CHUNK * _LANES,), jnp.float32),
            [pltpu.SemaphoreType.DMA] * _RING,
        ],
        compiler_params=pltpu.CompilerParams(needs_layout_passes=False),
    )
    def k(w_hbm, out_hbm, bufs, res, rowpart, sems):
        wid = lax.axis_index("s") * 2 + lax.axis_index("c")
        base = wid * _RPW * _EMBED
        def start(c):
            row0 = wid * _RPW + c * _CHUNK
            return pltpu.async_copy(
                w_hbm.at[pl.ds(row0, _CHUNK)], bufs[c % _RING],
                sems[c % _RING])

        lane = lax.iota(jnp.int32, _LANES)

        def shuf(v, idx):
            return v.at[idx].get(mode="promise_in_bounds",
                                 unique_indices=True)

        def combine(a, b, d):
            mask = (lane & d) == 0
            return (jnp.where(mask, a, shuf(b, lane ^ d))
                    + jnp.where(mask, shuf(a, lane ^ d), b))

        bitrev = (((lane & 1) << 3) | ((lane & 2) << 1)
                  | ((lane & 4) >> 1) | ((lane & 8) >> 3))

        cps = [start(c) for c in range(min(_RING, _NCHUNK))]
        for c in range(_NCHUNK):
            cps[c % _RING].wait()
            buf = bufs[c % _RING]
            @plsc.parallel_loop(0, _CHUNK, unroll=2)
            def _row(r, buf=buf):
                accs = [jnp.zeros((_LANES,), jnp.float32) for _ in range(4)]
                for k in range(_EMBED // _LANES):
                    accs[k % 4] = accs[k % 4] + buf[r, pl.ds(k * _LANES,
                                                             _LANES)]
                rowpart[pl.ds(r * _LANES, _LANES)] = (
                    (accs[0] + accs[1]) + (accs[2] + accs[3]))

            for g in range(_CHUNK // _LANES):
                cur = [rowpart[pl.ds((g * _LANES + i) * _LANES, _LANES)]
                       for i in range(_LANES)]
                for d in (8, 4, 2, 1):
                    cur = [combine(cur[2 * j], cur[2 * j + 1], d)
                           for j in range(len(cur) // 2)]
                res[pl.ds(c * _CHUNK + g * _LANES, _LANES)] = shuf(
                    cur[0], bitrev)
            if c + _RING < _NCHUNK:
                cps[c % _RING] = start(c + _RING)
        pltpu.sync_copy(res, out_hbm.at[pl.ds(wid * _RPW, _RPW)])

    return k(W_flat)


def _tc_rowsum_kernel(x_ref, W_ref, out_ref, S_acc):
    e = pl.program_id(0)

    @pl.when(e == 0)
    def _():
        S_acc[...] = jnp.zeros_like(S_acc)

    w_sum_e = jnp.sum(W_ref[0], axis=1)
    s_col = x_ref[...] @ w_sum_e[:, None]
    emask = (jax.lax.broadcasted_iota(jnp.int32, (1, _TC_E), 1) == e).astype(
        jnp.float32)
    S_acc[...] += s_col * emask

    @pl.when(e == _TC_E - 1)
    def _():
        out_ref[...] = S_acc[...]


def _combine_kernel(x_ref, Wg_ref, bg_ref, ws_ref, Stc_ref, b_ref, out_ref):
    logits = x_ref[...] @ Wg_ref[...] + bg_ref[...]
    m = jnp.max(logits, axis=1, keepdims=True)
    p = jnp.exp(logits - m)
    g = 1.0 / jnp.sum(p, axis=1)
    ii = jax.lax.broadcasted_iota(jnp.int32, (_B, _E), 1)
    idx = jnp.min(jnp.where(logits == m, ii, _E), axis=1)
    S_sc = lax.dot_general(x_ref[...], ws_ref[...],
                           (((1,), (1,)), ((), ())))
    S = jnp.concatenate([S_sc, Stc_ref[...]], axis=1)
    bsum = jnp.sum(b_ref[...], axis=1)
    A = g[:, None] * (S + bsum[None, :])
    H = (ii == idx[:, None]).astype(jnp.float32)
    out_ref[...] = A @ H.T


def kernel(x, Wg, bg, W, b):
    w_sum_sc = _rowsum_sc(
        W[:_SC_E].reshape(_ROWS, _EMBED)).reshape(_SC_E, _EMBED)
    S_tc = pl.pallas_call(
        _tc_rowsum_kernel,
        grid=(_TC_E,),
        in_specs=[
            pl.BlockSpec((_B, _EMBED), lambda e: (0, 0)),
            pl.BlockSpec((1, _EMBED, _EMBED), lambda e: (_SC_E + e, 0, 0)),
        ],
        out_specs=pl.BlockSpec((_B, _TC_E), lambda e: (0, 0)),
        out_shape=jax.ShapeDtypeStruct((_B, _TC_E), jnp.float32),
        scratch_shapes=[pltpu.VMEM((_B, _TC_E), jnp.float32)],
    )(x, W)
    return pl.pallas_call(
        _combine_kernel,
        out_shape=jax.ShapeDtypeStruct((_B, _B), jnp.float32),
    )(x, Wg, bg.reshape(1, _E), w_sum_sc, S_tc, b)


# speedup vs baseline: 1.2537x; 1.2537x over previous
import functools

import jax
import jax.numpy as jnp
from jax import lax
from jax.experimental import pallas as pl
from jax.experimental.pallas import tpu as pltpu
from jax.experimental.pallas import tpu_sc as plsc

_EMBED = 1024
_E = 16
_B = 128

_NW = 32
_SC_E = 16
_TC_E = _E - _SC_E
_ROWS = _SC_E * _EMBED
_RPW = _ROWS // _NW
_CHUNK = 32
_RING = 3
_NCHUNK = _RPW // _CHUNK
_LANES = 16


def _rowsum_sc(W_flat):
    mesh = plsc.VectorSubcoreMesh(core_axis_name="c", subcore_axis_name="s")

    @functools.partial(
        pl.kernel,
        mesh=mesh,
        out_type=jax.ShapeDtypeStruct((_ROWS,), jnp.float32),
        scratch_types=[
            pltpu.VMEM((_RING, _CHUNK, _EMBED), jnp.float32),
            pltpu.VMEM((_RPW,), jnp.float32),
            pltpu.VMEM((_CHUNK * _LANES,), jnp.float32),
            pltpu.SemaphoreType.DMA((_RING,)),
        ],
        compiler_params=pltpu.CompilerParams(needs_layout_passes=False),
    )
    def k(w_hbm, out_hbm, bufs, res, rowpart, sems):
        wid = lax.axis_index("s") * 2 + lax.axis_index("c")
        base = wid * _RPW * _EMBED
        def copy_desc(c):
            row0 = wid * _RPW + c * _CHUNK
            par = c % _RING
            return pltpu.make_async_copy(
                w_hbm.at[pl.ds(row0, _CHUNK)], bufs.at[par], sems.at[par])

        lane = lax.iota(jnp.int32, _LANES)

        def shuf(v, idx):
            return v.at[idx].get(mode="promise_in_bounds",
                                 unique_indices=True)

        def combine(a, b, d):
            mask = (lane & d) == 0
            return (jnp.where(mask, a, shuf(b, lane ^ d))
                    + jnp.where(mask, shuf(a, lane ^ d), b))

        bitrev = (((lane & 1) << 3) | ((lane & 2) << 1)
                  | ((lane & 4) >> 1) | ((lane & 8) >> 3))

        for c in range(min(_RING, _NCHUNK)):
            copy_desc(c).start()

        def chunk_body(c, carry):
            par = c % _RING
            copy_desc(c).wait()

            @plsc.parallel_loop(0, _CHUNK, unroll=2)
            def _row(r):
                accs = [jnp.zeros((_LANES,), jnp.float32) for _ in range(4)]
                for k in range(_EMBED // _LANES):
                    accs[k % 4] = accs[k % 4] + bufs[par, r,
                                                     pl.ds(k * _LANES,
                                                           _LANES)]
                rowpart[pl.ds(r * _LANES, _LANES)] = (
                    (accs[0] + accs[1]) + (accs[2] + accs[3]))

            for g in range(_CHUNK // _LANES):
                cur = [rowpart[pl.ds((g * _LANES + i) * _LANES, _LANES)]
                       for i in range(_LANES)]
                for d in (8, 4, 2, 1):
                    cur = [combine(cur[2 * j], cur[2 * j + 1], d)
                           for j in range(len(cur) // 2)]
                res[pl.ds(c * _CHUNK + g * _LANES, _LANES)] = shuf(
                    cur[0], bitrev)

            @pl.when(c + _RING < _NCHUNK)
            def _():
                copy_desc(c + _RING).start()
            return carry

        lax.fori_loop(0, _NCHUNK, chunk_body, jnp.int32(0))
        pltpu.sync_copy(res, out_hbm.at[pl.ds(wid * _RPW, _RPW)])

    return k(W_flat)


def _combine_kernel(x_ref, Wg_ref, bg_ref, ws_ref, b_ref, out_ref):
    logits = x_ref[...] @ Wg_ref[...] + bg_ref[...]
    m = jnp.max(logits, axis=1, keepdims=True)
    p = jnp.exp(logits - m)
    g = 1.0 / jnp.sum(p, axis=1)
    ii = jax.lax.broadcasted_iota(jnp.int32, (_B, _E), 1)
    idx = jnp.min(jnp.where(logits == m, ii, _E), axis=1)
    S = lax.dot_general(x_ref[...], ws_ref[...],
                        (((1,), (1,)), ((), ())))
    bsum = jnp.sum(b_ref[...], axis=1)
    A = g[:, None] * (S + bsum[None, :])
    H = (ii == idx[:, None]).astype(jnp.float32)
    out_ref[...] = A @ H.T


def kernel(x, Wg, bg, W, b):
    w_sum = _rowsum_sc(W.reshape(_ROWS, _EMBED)).reshape(_E, _EMBED)
    return pl.pallas_call(
        _combine_kernel,
        out_shape=jax.ShapeDtypeStruct((_B, _B), jnp.float32),
    )(x, Wg, bg.reshape(1, _E), w_sum, b)


# speedup vs baseline: 1.2647x; 1.0088x over previous
import functools

import jax
import jax.numpy as jnp
from jax import lax
from jax.experimental import pallas as pl
from jax.experimental.pallas import tpu as pltpu
from jax.experimental.pallas import tpu_sc as plsc

_EMBED = 1024
_E = 16
_B = 128

_NW = 32
_SC_E = 16
_TC_E = _E - _SC_E
_ROWS = _SC_E * _EMBED
_RPW = _ROWS // _NW
_CHUNK = 16
_RING = 6
_NCHUNK = _RPW // _CHUNK
_LANES = 16


def _rowsum_sc(W_flat):
    mesh = plsc.VectorSubcoreMesh(core_axis_name="c", subcore_axis_name="s")

    @functools.partial(
        pl.kernel,
        mesh=mesh,
        out_type=jax.ShapeDtypeStruct((_ROWS,), jnp.float32),
        scratch_types=[
            pltpu.VMEM((_RING, _CHUNK, _EMBED), jnp.float32),
            pltpu.VMEM((_RPW,), jnp.float32),
            pltpu.VMEM((_CHUNK * _LANES,), jnp.float32),
            pltpu.SemaphoreType.DMA((_RING,)),
        ],
        compiler_params=pltpu.CompilerParams(needs_layout_passes=False),
    )
    def k(w_hbm, out_hbm, bufs, res, rowpart, sems):
        wid = lax.axis_index("s") * 2 + lax.axis_index("c")
        base = wid * _RPW * _EMBED
        def copy_desc(c):
            row0 = wid * _RPW + c * _CHUNK
            par = c % _RING
            return pltpu.make_async_copy(
                w_hbm.at[pl.ds(row0, _CHUNK)], bufs.at[par], sems.at[par])

        lane = lax.iota(jnp.int32, _LANES)

        def shuf(v, idx):
            return v.at[idx].get(mode="promise_in_bounds",
                                 unique_indices=True)

        def combine(a, b, d):
            mask = (lane & d) == 0
            return (jnp.where(mask, a, shuf(b, lane ^ d))
                    + jnp.where(mask, shuf(a, lane ^ d), b))

        bitrev = (((lane & 1) << 3) | ((lane & 2) << 1)
                  | ((lane & 4) >> 1) | ((lane & 8) >> 3))

        for c in range(min(_RING, _NCHUNK)):
            copy_desc(c).start()

        def chunk_body(c, carry):
            par = c % _RING
            copy_desc(c).wait()

            @plsc.parallel_loop(0, _CHUNK, unroll=2)
            def _row(r):
                accs = [jnp.zeros((_LANES,), jnp.float32) for _ in range(4)]
                for k in range(_EMBED // _LANES):
                    accs[k % 4] = accs[k % 4] + bufs[par, r,
                                                     pl.ds(k * _LANES,
                                                           _LANES)]
                rowpart[pl.ds(r * _LANES, _LANES)] = (
                    (accs[0] + accs[1]) + (accs[2] + accs[3]))

            for g in range(_CHUNK // _LANES):
                cur = [rowpart[pl.ds((g * _LANES + i) * _LANES, _LANES)]
                       for i in range(_LANES)]
                for d in (8, 4, 2, 1):
                    cur = [combine(cur[2 * j], cur[2 * j + 1], d)
                           for j in range(len(cur) // 2)]
                res[pl.ds(c * _CHUNK + g * _LANES, _LANES)] = shuf(
                    cur[0], bitrev)

            @pl.when(c + _RING < _NCHUNK)
            def _():
                copy_desc(c + _RING).start()
            return carry

        lax.fori_loop(0, _NCHUNK, chunk_body, jnp.int32(0))
        pltpu.sync_copy(res, out_hbm.at[pl.ds(wid * _RPW, _RPW)])

    return k(W_flat)


def _combine_kernel(x_ref, Wg_ref, bg_ref, ws_ref, b_ref, out_ref):
    logits = x_ref[...] @ Wg_ref[...] + bg_ref[...]
    m = jnp.max(logits, axis=1, keepdims=True)
    p = jnp.exp(logits - m)
    g = 1.0 / jnp.sum(p, axis=1)
    ii = jax.lax.broadcasted_iota(jnp.int32, (_B, _E), 1)
    idx = jnp.min(jnp.where(logits == m, ii, _E), axis=1)
    S = lax.dot_general(x_ref[...], ws_ref[...],
                        (((1,), (1,)), ((), ())))
    bsum = jnp.sum(b_ref[...], axis=1)
    A = g[:, None] * (S + bsum[None, :])
    H = (ii == idx[:, None]).astype(jnp.float32)
    out_ref[...] = A @ H.T


def kernel(x, Wg, bg, W, b):
    w_sum = _rowsum_sc(W.reshape(_ROWS, _EMBED)).reshape(_E, _EMBED)
    return pl.pallas_call(
        _combine_kernel,
        out_shape=jax.ShapeDtypeStruct((_B, _B), jnp.float32),
    )(x, Wg, bg.reshape(1, _E), w_sum, b)


# speedup vs baseline: 1.2713x; 1.0052x over previous
import functools

import jax
import jax.numpy as jnp
from jax import lax
from jax.experimental import pallas as pl
from jax.experimental.pallas import tpu as pltpu
from jax.experimental.pallas import tpu_sc as plsc

_EMBED = 1024
_E = 16
_B = 128

_NW = 32
_ROWS = _E * _EMBED
_RPW = _ROWS // _NW
_CHUNK = 16
_RING = 6
_NCHUNK = _RPW // _CHUNK
_LANES = 16


def _rowsum_sc(W_flat):
    mesh = plsc.VectorSubcoreMesh(core_axis_name="c", subcore_axis_name="s")

    @functools.partial(
        pl.kernel,
        mesh=mesh,
        out_type=jax.ShapeDtypeStruct((_ROWS,), jnp.float32),
        scratch_types=[
            pltpu.VMEM((_RING, _CHUNK, _EMBED), jnp.float32),
            pltpu.VMEM((_RPW,), jnp.float32),
            pltpu.VMEM((_CHUNK * _LANES,), jnp.float32),
            pltpu.SemaphoreType.DMA((_RING,)),
        ],
        compiler_params=pltpu.CompilerParams(needs_layout_passes=False),
    )
    def k(w_hbm, out_hbm, bufs, res, rowpart, sems):
        wid = lax.axis_index("s") * 2 + lax.axis_index("c")

        def copy_desc(c):
            row0 = wid * _RPW + c * _CHUNK
            par = c % _RING
            return pltpu.make_async_copy(
                w_hbm.at[pl.ds(row0, _CHUNK)], bufs.at[par], sems.at[par])

        lane = lax.iota(jnp.int32, _LANES)

        def shuf(v, idx):
            return v.at[idx].get(mode="promise_in_bounds",
                                 unique_indices=True)

        def combine(a, b, d):
            mask = (lane & d) == 0
            return (jnp.where(mask, a, shuf(b, lane ^ d))
                    + jnp.where(mask, shuf(a, lane ^ d), b))

        bitrev = (((lane & 1) << 3) | ((lane & 2) << 1)
                  | ((lane & 4) >> 1) | ((lane & 8) >> 3))

        for c in range(min(_RING, _NCHUNK)):
            copy_desc(c).start()

        def chunk_body(c, carry):
            par = c % _RING
            copy_desc(c).wait()

            @plsc.parallel_loop(0, _CHUNK, unroll=2)
            def _row(r):
                accs = [jnp.zeros((_LANES,), jnp.float32) for _ in range(4)]
                for k in range(_EMBED // _LANES):
                    accs[k % 4] = accs[k % 4] + bufs[par, r,
                                                     pl.ds(k * _LANES,
                                                           _LANES)]
                rowpart[pl.ds(r * _LANES, _LANES)] = (
                    (accs[0] + accs[1]) + (accs[2] + accs[3]))

            for g in range(_CHUNK // _LANES):
                cur = [rowpart[pl.ds((g * _LANES + i) * _LANES, _LANES)]
                       for i in range(_LANES)]
                for d in (8, 4, 2, 1):
                    cur = [combine(cur[2 * j], cur[2 * j + 1], d)
                           for j in range(len(cur) // 2)]
                res[pl.ds(c * _CHUNK + g * _LANES, _LANES)] = shuf(
                    cur[0], bitrev)

            @pl.when(c + _RING < _NCHUNK)
            def _():
                copy_desc(c + _RING).start()
            return carry

        lax.fori_loop(0, _NCHUNK, chunk_body, jnp.int32(0))
        pltpu.sync_copy(res, out_hbm.at[pl.ds(wid * _RPW, _RPW)])

    return k(W_flat)


def _combine_kernel(x_ref, Wg_ref, bg_ref, ws_ref, b_ref, out_ref):
    logits = x_ref[...] @ Wg_ref[...] + bg_ref[...]
    m = jnp.max(logits, axis=1, keepdims=True)
    p = jnp.exp(logits - m)
    g = 1.0 / jnp.sum(p, axis=1)
    ii = jax.lax.broadcasted_iota(jnp.int32, (_B, _E), 1)
    idx = jnp.min(jnp.where(logits == m, ii, _E), axis=1)
    S = lax.dot_general(x_ref[...], ws_ref[...],
                        (((1,), (1,)), ((), ())))
    bsum = jnp.sum(b_ref[...], axis=1)
    A = g[:, None] * (S + bsum[None, :])
    H = (ii == idx[:, None]).astype(jnp.float32)
    out_ref[...] = A @ H.T


def kernel(x, Wg, bg, W, b):
    w_sum = _rowsum_sc(W.reshape(_ROWS, _EMBED)).reshape(_E, _EMBED)
    return pl.pallas_call(
        _combine_kernel,
        out_shape=jax.ShapeDtypeStruct((_B, _B), jnp.float32),
    )(x, Wg, bg.reshape(1, _E), w_sum, b)
